# baseline (speedup 1.0000x reference)
import functools

import jax
import jax.numpy as jnp
from jax.experimental import pallas as pl
from jax.experimental.pallas import tpu as pltpu


def _seq_query_block(e_ref, q_ref, w1_ref, w2_ref, b1_ref, b2_ref, aw_ref,
                     ab_ref, out_ref, *, seg_per_block, seg_len):
    blk = pl.program_id(0)
    e = e_ref[...]
    q = q_ref[pl.ds(blk * seg_per_block, seg_per_block), :]
    qw = jax.lax.dot_general(q, w1_ref[...], (((1,), (1,)), ((), ())),
                             preferred_element_type=jnp.float32)
    qw = qw + b1_ref[...] + b2_ref[...]
    z = jax.lax.dot_general(e.astype(jnp.bfloat16),
                            w2_ref[...].astype(jnp.bfloat16),
                            (((1,), (1,)), ((), ())),
                            preferred_element_type=jnp.float32)
    rows = []
    for s in range(seg_per_block):
        lo = s * seg_len
        hs = 0.5 * jnp.tanh((z[lo:lo + seg_len] + qw[s:s + 1]) * 0.5) + 0.5
        es = e[lo:lo + seg_len]
        g = jax.lax.dot_general(hs, es, (((0,), (0,)), ((), ())),
                                preferred_element_type=jnp.float32)
        esum = jnp.sum(es, axis=0, keepdims=True)
        rows.append(
            jnp.dot(aw_ref[...], g, preferred_element_type=jnp.float32)
            + ab_ref[0, 0] * esum)
    out_ref[pl.ds(blk * seg_per_block, seg_per_block), :] = (
        jnp.concatenate(rows, axis=0))


def kernel(sess_embed, query, W1_w, W1_b, W2_w, W2_b, alpha_w, alpha_b,
           sections):
    N, d = sess_embed.shape
    B = query.shape[0]
    S = N // B
    SB = 4
    body = functools.partial(_seq_query_block, seg_per_block=SB, seg_len=S)

    return pl.pallas_call(
        body,
        grid=(B // SB,),
        in_specs=[
            pl.BlockSpec((SB * S, d), lambda b: (b, 0)),
            pl.BlockSpec((B, d), lambda b: (0, 0)),
            pl.BlockSpec((d, d), lambda b: (0, 0)),
            pl.BlockSpec((d, d), lambda b: (0, 0)),
            pl.BlockSpec((1, d), lambda b: (0, 0)),
            pl.BlockSpec((1, d), lambda b: (0, 0)),
            pl.BlockSpec((1, d), lambda b: (0, 0)),
            pl.BlockSpec((1, 1), lambda b: (0, 0)),
        ],
        out_specs=pl.BlockSpec((B, d), lambda b: (0, 0)),
        out_shape=jax.ShapeDtypeStruct((B, d), jnp.float32),
        compiler_params=pltpu.CompilerParams(
            dimension_semantics=("arbitrary",)),
    )(sess_embed, query, W1_w, W2_w, W1_b.reshape(1, d), W2_b.reshape(1, d),
      alpha_w, alpha_b.reshape(1, 1))


# speedup vs baseline: 1.0325x; 1.0325x over previous
import functools

import jax
import jax.numpy as jnp
from jax.experimental import pallas as pl
from jax.experimental.pallas import tpu as pltpu


def _seq_query_block(e_ref, q_ref, w1_ref, w2_ref, b1_ref, b2_ref, aw_ref,
                     ab_ref, out_ref, *, seg_per_block, seg_len):
    blk = pl.program_id(0)
    e = e_ref[...]
    q = q_ref[pl.ds(blk * seg_per_block, seg_per_block), :]
    qw = jax.lax.dot_general(q, w1_ref[...], (((1,), (1,)), ((), ())),
                             preferred_element_type=jnp.float32)
    qw = qw + b1_ref[...] + b2_ref[...]
    d = qw.shape[1]
    w2 = w2_ref[...]
    zpad = jnp.zeros((d, d), dtype=jnp.float32)
    w2bd = jnp.concatenate(
        [jnp.concatenate([w2, zpad], axis=1),
         jnp.concatenate([zpad, w2], axis=1)], axis=0)
    aw = aw_ref[...]
    ab = ab_ref[0, 0]
    rows = [None] * seg_per_block
    for s in range(0, seg_per_block, 2):
        lo = s * seg_len
        ep = jnp.concatenate([e[lo:lo + seg_len],
                              e[lo + seg_len:lo + 2 * seg_len]],
                             axis=1)
        qp = jnp.concatenate([qw[s:s + 1], qw[s + 1:s + 2]], axis=1)
        zp = jax.lax.dot_general(ep, w2bd, (((1,), (1,)), ((), ())),
                                 preferred_element_type=jnp.float32)
        hp = 0.5 * jnp.tanh((zp + qp) * 0.5) + 0.5
        gp = jax.lax.dot_general(hp, ep, (((0,), (0,)), ((), ())),
                                 preferred_element_type=jnp.float32)
        esp = jnp.sum(ep, axis=0, keepdims=True)
        rows[s] = (jnp.dot(aw, gp[:d, :d], preferred_element_type=jnp.float32)
                   + ab * esp[:, :d])
        rows[s + 1] = (jnp.dot(aw, gp[d:, d:],
                               preferred_element_type=jnp.float32)
                       + ab * esp[:, d:])
    out_ref[pl.ds(blk * seg_per_block, seg_per_block), :] = (
        jnp.concatenate(rows, axis=0))


def kernel(sess_embed, query, W1_w, W1_b, W2_w, W2_b, alpha_w, alpha_b,
           sections):
    N, d = sess_embed.shape
    B = query.shape[0]
    S = N // B
    SB = 4
    body = functools.partial(_seq_query_block, seg_per_block=SB, seg_len=S)

    return pl.pallas_call(
        body,
        grid=(B // SB,),
        in_specs=[
            pl.BlockSpec((SB * S, d), lambda b: (b, 0)),
            pl.BlockSpec((B, d), lambda b: (0, 0)),
            pl.BlockSpec((d, d), lambda b: (0, 0)),
            pl.BlockSpec((d, d), lambda b: (0, 0)),
            pl.BlockSpec((1, d), lambda b: (0, 0)),
            pl.BlockSpec((1, d), lambda b: (0, 0)),
            pl.BlockSpec((1, d), lambda b: (0, 0)),
            pl.BlockSpec((1, 1), lambda b: (0, 0)),
        ],
        out_specs=pl.BlockSpec((B, d), lambda b: (0, 0)),
        out_shape=jax.ShapeDtypeStruct((B, d), jnp.float32),
        compiler_params=pltpu.CompilerParams(
            dimension_semantics=("arbitrary",)),
    )(sess_embed, query, W1_w, W2_w, W1_b.reshape(1, d), W2_b.reshape(1, d),
      alpha_w, alpha_b.reshape(1, 1))


# speedup vs baseline: 1.0351x; 1.0025x over previous
import functools

import jax
import jax.numpy as jnp
from jax.experimental import pallas as pl
from jax.experimental.pallas import tpu as pltpu


def _seq_query_block(e_ref, q_ref, w1_ref, w2_ref, b1_ref, b2_ref, aw_ref,
                     ab_ref, out_ref, *, seg_per_block, seg_len):
    blk = pl.program_id(0)
    q = q_ref[pl.ds(blk * seg_per_block, seg_per_block), :]
    qw = jax.lax.dot_general(q, w1_ref[...], (((1,), (1,)), ((), ())),
                             preferred_element_type=jnp.float32)
    qw = qw + b1_ref[...] + b2_ref[...]
    d = qw.shape[1]
    w2 = w2_ref[...]
    zpad = jnp.zeros((d, d), dtype=jnp.float32)
    w2bd = jnp.concatenate(
        [jnp.concatenate([w2, zpad], axis=1),
         jnp.concatenate([zpad, w2], axis=1)], axis=0)
    aw = aw_ref[...]
    ab = ab_ref[0, 0]
    rows = [None] * seg_per_block
    for s in range(0, seg_per_block, 2):
        lo = s * seg_len
        ep = jnp.concatenate([e_ref[lo:lo + seg_len, :],
                              e_ref[lo + seg_len:lo + 2 * seg_len, :]],
                             axis=1)
        qp = jnp.concatenate([qw[s:s + 1], qw[s + 1:s + 2]], axis=1)
        zp = jax.lax.dot_general(ep, w2bd, (((1,), (1,)), ((), ())),
                                 preferred_element_type=jnp.float32)
        hp = 0.5 * jnp.tanh((zp + qp) * 0.5) + 0.5
        gp = jax.lax.dot_general(hp, ep, (((0,), (0,)), ((), ())),
                                 preferred_element_type=jnp.float32)
        esp = jnp.sum(ep, axis=0, keepdims=True)
        rows[s] = (jnp.dot(aw, gp[:d, :d], preferred_element_type=jnp.float32)
                   + ab * esp[:, :d])
        rows[s + 1] = (jnp.dot(aw, gp[d:, d:],
                               preferred_element_type=jnp.float32)
                       + ab * esp[:, d:])
    out_ref[pl.ds(blk * seg_per_block, seg_per_block), :] = (
        jnp.concatenate(rows, axis=0))


def kernel(sess_embed, query, W1_w, W1_b, W2_w, W2_b, alpha_w, alpha_b,
           sections):
    N, d = sess_embed.shape
    B = query.shape[0]
    S = N // B
    SB = 4
    body = functools.partial(_seq_query_block, seg_per_block=SB, seg_len=S)

    return pl.pallas_call(
        body,
        grid=(B // SB,),
        in_specs=[
            pl.BlockSpec((SB * S, d), lambda b: (b, 0)),
            pl.BlockSpec((B, d), lambda b: (0, 0)),
            pl.BlockSpec((d, d), lambda b: (0, 0)),
            pl.BlockSpec((d, d), lambda b: (0, 0)),
            pl.BlockSpec((1, d), lambda b: (0, 0)),
            pl.BlockSpec((1, d), lambda b: (0, 0)),
            pl.BlockSpec((1, d), lambda b: (0, 0)),
            pl.BlockSpec((1, 1), lambda b: (0, 0)),
        ],
        out_specs=pl.BlockSpec((B, d), lambda b: (0, 0)),
        out_shape=jax.ShapeDtypeStruct((B, d), jnp.float32),
        compiler_params=pltpu.CompilerParams(
            dimension_semantics=("arbitrary",)),
    )(sess_embed, query, W1_w, W2_w, W1_b.reshape(1, d), W2_b.reshape(1, d),
      alpha_w, alpha_b.reshape(1, 1))
